# Initial kernel scaffold; baseline (speedup 1.0000x reference)
import jax
import jax.numpy as jnp
from jax.experimental import pallas as pl


def kernel(x, edge_index, c1w1, c1b1, c1w2, c1b2, c2w1, c2b1, c2w2, c2b2, c3w1, c3b1, c3w2, c3b2, l1w, l1b, l2w, l2b):
    raise NotImplementedError("write your pallas kernel here")


# speedup vs baseline: 8.4039x; 8.4039x over previous
import functools

import jax
import jax.numpy as jnp
from jax import lax
from jax.experimental import pallas as pl
from jax.experimental.pallas import tpu as pltpu
from jax.experimental.pallas import tpu_sc as plsc

N = 10000
E = 320000
D = 128

NC = 2
NS = 16
NW = NC * NS
EPW = E // NW
CH = 125
NCHUNK = EPW // CH
RPS = 624
RPS_LAST = N - 15 * RPS


def _sc_aggregate(x, edges_r, zeros_nd):
    @functools.partial(
        pl.kernel,
        out_type=jax.ShapeDtypeStruct((NC, N, D), jnp.float32),
        mesh=plsc.VectorSubcoreMesh(core_axis_name="c", subcore_axis_name="s"),
        scratch_types=[
            pltpu.VMEM((NCHUNK, CH), jnp.int32),
            pltpu.VMEM((NCHUNK, CH), jnp.int32),
            pltpu.VMEM((CH, D), jnp.float32),
            pltpu.VMEM_SHARED((N, D), jnp.float32),
            pltpu.SemaphoreType.DMA,
        ],
    )
    def agg(x_hbm, e_hbm, z_hbm, out_hbm, src_v, dst_v, rows_v, acc_sh, sem):
        c = lax.axis_index("c")
        s = lax.axis_index("s")
        wid = s * NC + c
        r0 = s * RPS

        @pl.when(s < NS - 1)
        def _():
            pltpu.sync_copy(z_hbm.at[pl.ds(r0, RPS)], acc_sh.at[pl.ds(r0, RPS)])

        @pl.when(s == NS - 1)
        def _():
            pltpu.sync_copy(z_hbm.at[pl.ds(15 * RPS, RPS_LAST)],
                            acc_sh.at[pl.ds(15 * RPS, RPS_LAST)])

        pltpu.sync_copy(e_hbm.at[0, wid], src_v)
        pltpu.sync_copy(e_hbm.at[1, wid], dst_v)
        plsc.subcore_barrier()

        def body(ch, carry):
            pltpu.async_copy(x_hbm.at[src_v.at[ch]], rows_v, sem).wait()
            pltpu.sync_copy(rows_v, acc_sh.at[dst_v.at[ch]], add=True)
            return carry

        lax.fori_loop(0, NCHUNK, body, 0)
        plsc.subcore_barrier()

        @pl.when(s < NS - 1)
        def _():
            pltpu.sync_copy(acc_sh.at[pl.ds(r0, RPS)],
                            out_hbm.at[c, pl.ds(r0, RPS)])

        @pl.when(s == NS - 1)
        def _():
            pltpu.sync_copy(acc_sh.at[pl.ds(15 * RPS, RPS_LAST)],
                            out_hbm.at[c, pl.ds(15 * RPS, RPS_LAST)])

    return agg(x, edges_r, zeros_nd)


_BN = 1000


def _tc_layer_body(x_ref, a_ref, w1_ref, b1_ref, w2_ref, b2_ref, o_ref):
    h = x_ref[...] + a_ref[0] + a_ref[1]
    h = jnp.maximum(jnp.dot(h, w1_ref[...], preferred_element_type=jnp.float32)
                    + b1_ref[...], 0.0)
    h = jnp.maximum(jnp.dot(h, w2_ref[...], preferred_element_type=jnp.float32)
                    + b2_ref[...], 0.0)
    o_ref[...] = h


def _tc_final_body(x_ref, a_ref, w1_ref, b1_ref, w2_ref, b2_ref,
                   l1w_ref, l1b_ref, l2w_ref, l2b_ref, o_ref):
    h = x_ref[...] + a_ref[0] + a_ref[1]
    h = jnp.maximum(jnp.dot(h, w1_ref[...], preferred_element_type=jnp.float32)
                    + b1_ref[...], 0.0)
    h = jnp.maximum(jnp.dot(h, w2_ref[...], preferred_element_type=jnp.float32)
                    + b2_ref[...], 0.0)
    h = jnp.maximum(jnp.dot(h, l1w_ref[...], preferred_element_type=jnp.float32)
                    + l1b_ref[...], 0.0)
    o_ref[...] = (jnp.dot(h, l2w_ref[...], preferred_element_type=jnp.float32)
                  + l2b_ref[...])


def _row_spec():
    return pl.BlockSpec((_BN, D), lambda i: (i, 0))


def _agg_spec():
    return pl.BlockSpec((NC, _BN, D), lambda i: (0, i, 0))


def _w_spec():
    return pl.BlockSpec((D, D), lambda i: (0, 0))


def _b_spec():
    return pl.BlockSpec((1, D), lambda i: (0, 0))


def _tc_layer(x, agg, w1, b1, w2, b2):
    return pl.pallas_call(
        _tc_layer_body,
        grid=(N // _BN,),
        in_specs=[_row_spec(), _agg_spec(), _w_spec(), _b_spec(), _w_spec(),
                  _b_spec()],
        out_specs=_row_spec(),
        out_shape=jax.ShapeDtypeStruct((N, D), jnp.float32),
    )(x, agg, w1, b1.reshape(1, D), w2, b2.reshape(1, D))


def _tc_final(x, agg, w1, b1, w2, b2, l1w, l1b, l2w, l2b):
    return pl.pallas_call(
        _tc_final_body,
        grid=(N // _BN,),
        in_specs=[_row_spec(), _agg_spec(), _w_spec(), _b_spec(), _w_spec(),
                  _b_spec(), _w_spec(), _b_spec(), _w_spec(), _b_spec()],
        out_specs=_row_spec(),
        out_shape=jax.ShapeDtypeStruct((N, D), jnp.float32),
    )(x, agg, w1, b1.reshape(1, D), w2, b2.reshape(1, D),
      l1w, l1b.reshape(1, D), l2w, l2b.reshape(1, D))


def kernel(x, edge_index, c1w1, c1b1, c1w2, c1b2, c2w1, c2b1, c2w2, c2b2,
           c3w1, c3b1, c3w2, c3b2, l1w, l1b, l2w, l2b):
    edges_r = edge_index.reshape(2, NW, NCHUNK, CH)
    z = jnp.zeros((N, D), jnp.float32)
    a = _sc_aggregate(x, edges_r, z)
    h = _tc_layer(x, a, c1w1, c1b1, c1w2, c1b2)
    a = _sc_aggregate(h, edges_r, z)
    h = _tc_layer(h, a, c2w1, c2b1, c2w2, c2b2)
    a = _sc_aggregate(h, edges_r, z)
    return _tc_final(h, a, c3w1, c3b1, c3w2, c3b2, l1w, l1b, l2w, l2b)
